# baseline (speedup 1.0000x reference)
import functools

import jax
import jax.numpy as jnp
from jax import lax
from jax.experimental import pallas as pl
from jax.experimental.pallas import tpu as pltpu
from jax.experimental.pallas import tpu_sc as plsc


def _logits_body(x_ref, sim_ref, gates_ref, lg_ref, xb_ref):
    x = x_ref[...]
    sim = sim_ref[...]
    g = gates_ref[...]
    rn = jnp.sqrt(jnp.sum(x * x, axis=1, keepdims=True))
    hn = x / jnp.maximum(rn, 1e-12)
    cn = jnp.sqrt(jnp.sum(sim * sim, axis=0, keepdims=True))
    sn = sim / jnp.maximum(cn, 1e-12)
    sig = 1.0 / (1.0 + jnp.exp(-g))
    lg = jnp.dot(hn, sn, preferred_element_type=jnp.float32) - sig
    lg_ref[...] = lg.T
    xb_ref[...] = x.astype(jnp.bfloat16)


def _sc_route_body(lg_hbm, am_hbm, w_hbm, lv, amv, wv, *, TPW, E, NC):
    wid = lax.axis_index("s") * NC + lax.axis_index("c")
    base = wid * TPW
    pltpu.sync_copy(lg_hbm.at[:, pl.ds(base, TPW)], lv)
    for gidx in range(TPW // 16):
        sl = pl.ds(gidx * 16, 16)
        ls = [lv[e, sl] for e in range(E)]
        gated = [jnp.maximum(l, 0.0) for l in ls]
        mask = [(l > 0.0).astype(jnp.float32) for l in ls]
        npos = functools.reduce(lambda a, b: a + b, mask)
        m1 = functools.reduce(jnp.maximum, ls)
        cand1 = [jnp.where(ls[e] == m1, e, E) for e in range(E)]
        i1 = functools.reduce(jnp.minimum, cand1)
        l2 = [jnp.where(i1 == e, -jnp.float32(1e30), ls[e]) for e in range(E)]
        m2 = functools.reduce(jnp.maximum, l2)
        cand2 = [jnp.where(l2[e] == m2, e, E) for e in range(E)]
        i2 = functools.reduce(jnp.minimum, cand2)
        inact = npos == 0.0
        am = [jnp.where(inact & ((i1 == e) | (i2 == e)), 1.0, mask[e])
              for e in range(E)]
        gm = [jnp.where(am[e] > 0.0, gated[e], -1e9) for e in range(E)]
        mx = functools.reduce(jnp.maximum, gm)
        ex = [jnp.exp(gm[e] - mx) for e in range(E)]
        ssum = functools.reduce(lambda a, b: a + b, ex)
        sinv = 1.0 / ssum
        for e in range(E):
            amv[e, sl] = am[e]
            wv[e, sl] = ex[e] * sinv * am[e]
    pltpu.sync_copy(amv, am_hbm.at[:, pl.ds(base, TPW)])
    pltpu.sync_copy(wv, w_hbm.at[:, pl.ds(base, TPW)])


def _qkv_body(x_ref, am_ref, wqkv_ref, rep_ref, q_ref, k_ref, v_ref, *, E, HD):
    x = x_ref[...]
    am = am_ref[...].T
    BN = x.shape[0]
    p = jnp.dot(x, wqkv_ref[...], preferred_element_type=jnp.float32)
    qk = jnp.zeros((BN, 2 * HD), jnp.float32)
    for i in range(E):
        qk = qk + am[:, i:i + 1] * p[:, i * 2 * HD:(i + 1) * 2 * HD]
    q_ref[...] = qk[:, :HD]
    k_ref[...] = qk[:, HD:]
    amr = jnp.dot(am.astype(jnp.bfloat16), rep_ref[...],
                  preferred_element_type=jnp.float32)
    voff = 2 * E * HD
    accv = jnp.zeros((BN, 2 * HD), jnp.float32)
    for j in range(E // 2):
        sl = slice(voff + j * 2 * HD, voff + (j + 1) * 2 * HD)
        accv = accv + p[:, sl] * amr[:, j * 2 * HD:(j + 1) * 2 * HD]
    v_ref[...] = accv[:, :HD] + accv[:, HD:]


def _attn_out_body(q_ref, k_ref, v_ref, w_ref, o_ref, out_ref, *,
                   BQ, T, E, HD, scale):
    qb = pl.program_id(1)
    q = q_ref[...].astype(jnp.bfloat16)
    k = k_ref[...].astype(jnp.bfloat16)
    s = lax.dot_general(q, k, (((1,), (1,)), ((), ())),
                        preferred_element_type=jnp.float32)
    rows = qb * BQ + lax.broadcasted_iota(jnp.int32, (BQ, T), 0)
    cols = lax.broadcasted_iota(jnp.int32, (BQ, T), 1)
    s = jnp.where(cols <= rows, s * scale, -1e9)
    m = jnp.max(s, axis=1, keepdims=True)
    p = jnp.exp(s - m)
    l = jnp.sum(p, axis=1, keepdims=True)
    oh = jnp.dot(p.astype(jnp.bfloat16), v_ref[...].astype(jnp.bfloat16),
                 preferred_element_type=jnp.float32)
    oh = oh / l

    w = w_ref[...].T
    a2 = jnp.concatenate([oh * w[:, i:i + 1] for i in range(E)], axis=1)
    out_ref[...] = jnp.dot(a2.astype(jnp.bfloat16), o_ref[...],
                           preferred_element_type=jnp.float32)


def kernel(hidden_states, sim_matrix, gates, q_proj, k_proj, v_proj, o_proj):
    B, T, C = hidden_states.shape
    E = sim_matrix.shape[1]
    HD = q_proj.shape[2]
    N = B * T
    flat = hidden_states.reshape(N, C)

    wqk = jnp.concatenate([q_proj, k_proj], axis=2)
    wqk = wqk.transpose(1, 0, 2).reshape(C, E * 2 * HD)
    wv = v_proj.transpose(1, 0, 2).reshape(C, E * HD)
    wqkv = jnp.concatenate([wqk, wv], axis=1).astype(jnp.bfloat16)
    rep = (jnp.arange(E * HD)[None, :] // HD
           == jnp.arange(E)[:, None]).astype(jnp.bfloat16)
    o_stack = o_proj.reshape(E * HD, C).astype(jnp.bfloat16)
    gates_row = gates.reshape(1, E)

    BN = 1024 if N % 1024 == 0 else N
    g1 = N // BN
    logits, flat_bf = pl.pallas_call(
        _logits_body,
        grid=(g1,),
        in_specs=[
            pl.BlockSpec((BN, C), lambda i: (i, 0)),
            pl.BlockSpec((C, E), lambda i: (0, 0)),
            pl.BlockSpec((1, E), lambda i: (0, 0)),
        ],
        out_specs=[
            pl.BlockSpec((E, BN), lambda i: (0, i)),
            pl.BlockSpec((BN, C), lambda i: (i, 0)),
        ],
        out_shape=[
            jax.ShapeDtypeStruct((E, N), jnp.float32),
            jax.ShapeDtypeStruct((N, C), jnp.bfloat16),
        ],
    )(flat, sim_matrix, gates_row)

    NC, NS = 2, 16
    TPW = N // (NC * NS)
    route = pl.kernel(
        functools.partial(_sc_route_body, TPW=TPW, E=E, NC=NC),
        out_type=[jax.ShapeDtypeStruct((E, N), jnp.float32),
                  jax.ShapeDtypeStruct((E, N), jnp.float32)],
        mesh=plsc.VectorSubcoreMesh(core_axis_name="c", subcore_axis_name="s"),
        scratch_types=[pltpu.VMEM((E, TPW), jnp.float32)] * 3,
        compiler_params=pltpu.CompilerParams(needs_layout_passes=False),
    )
    am_t, w_t = route(logits)

    q, k, v = pl.pallas_call(
        functools.partial(_qkv_body, E=E, HD=HD),
        grid=(g1,),
        in_specs=[
            pl.BlockSpec((BN, C), lambda i: (i, 0)),
            pl.BlockSpec((E, BN), lambda i: (0, i)),
            pl.BlockSpec((C, E * 3 * HD), lambda i: (0, 0)),
            pl.BlockSpec((E, E * HD), lambda i: (0, 0)),
        ],
        out_specs=[
            pl.BlockSpec((BN, HD), lambda i: (i, 0)),
            pl.BlockSpec((BN, HD), lambda i: (i, 0)),
            pl.BlockSpec((BN, HD), lambda i: (i, 0)),
        ],
        out_shape=[
            jax.ShapeDtypeStruct((N, HD), jnp.float32),
            jax.ShapeDtypeStruct((N, HD), jnp.float32),
            jax.ShapeDtypeStruct((N, HD), jnp.float32),
        ],
    )(flat_bf, am_t, wqkv, rep)

    qb3 = q.reshape(B, T, HD)
    kb3 = k.reshape(B, T, HD)
    vb3 = v.reshape(B, T, HD)
    wb3 = w_t.reshape(E, B, T).transpose(1, 0, 2)

    BQ = 256 if T % 256 == 0 else T
    scale = 1.0 / float(HD) ** 0.5
    out = pl.pallas_call(
        functools.partial(_attn_out_body, BQ=BQ, T=T, E=E, HD=HD,
                          scale=scale),
        grid=(B, T // BQ),
        in_specs=[
            pl.BlockSpec((None, BQ, HD), lambda b, i: (b, i, 0)),
            pl.BlockSpec((None, T, HD), lambda b, i: (b, 0, 0)),
            pl.BlockSpec((None, T, HD), lambda b, i: (b, 0, 0)),
            pl.BlockSpec((None, E, BQ), lambda b, i: (b, 0, i)),
            pl.BlockSpec((E * HD, C), lambda b, i: (0, 0)),
        ],
        out_specs=pl.BlockSpec((None, BQ, C), lambda b, i: (b, i, 0)),
        out_shape=jax.ShapeDtypeStruct((B, T, C), jnp.float32),
    )(qb3, kb3, vb3, wb3, o_stack)
    return out


# speedup vs baseline: 1.0345x; 1.0345x over previous
import functools

import jax
import jax.numpy as jnp
from jax import lax
from jax.experimental import pallas as pl
from jax.experimental.pallas import tpu as pltpu
from jax.experimental.pallas import tpu_sc as plsc


def _logits_body(x_ref, sim_ref, gates_ref, lg_ref):
    x = x_ref[...]
    sim = sim_ref[...]
    g = gates_ref[...]
    rn = jnp.sqrt(jnp.sum(x * x, axis=1, keepdims=True))
    hn = x / jnp.maximum(rn, 1e-12)
    cn = jnp.sqrt(jnp.sum(sim * sim, axis=0, keepdims=True))
    sn = sim / jnp.maximum(cn, 1e-12)
    sig = 1.0 / (1.0 + jnp.exp(-g))
    lg = jnp.dot(hn, sn, preferred_element_type=jnp.float32) - sig
    lg_ref[...] = lg.T


def _sc_route_body(lg_hbm, am_hbm, w_hbm, lv, amv, wv, *, TPW, E, NC):
    wid = lax.axis_index("s") * NC + lax.axis_index("c")
    base = wid * TPW
    pltpu.sync_copy(lg_hbm.at[:, pl.ds(base, TPW)], lv)
    for gidx in range(TPW // 16):
        sl = pl.ds(gidx * 16, 16)
        ls = [lv[e, sl] for e in range(E)]
        gated = [jnp.maximum(l, 0.0) for l in ls]
        mask = [(l > 0.0).astype(jnp.float32) for l in ls]
        npos = functools.reduce(lambda a, b: a + b, mask)
        m1 = functools.reduce(jnp.maximum, ls)
        cand1 = [jnp.where(ls[e] == m1, e, E) for e in range(E)]
        i1 = functools.reduce(jnp.minimum, cand1)
        l2 = [jnp.where(i1 == e, -jnp.float32(1e30), ls[e]) for e in range(E)]
        m2 = functools.reduce(jnp.maximum, l2)
        cand2 = [jnp.where(l2[e] == m2, e, E) for e in range(E)]
        i2 = functools.reduce(jnp.minimum, cand2)
        inact = npos == 0.0
        am = [jnp.where(inact & ((i1 == e) | (i2 == e)), 1.0, mask[e])
              for e in range(E)]
        gm = [jnp.where(am[e] > 0.0, gated[e], -1e9) for e in range(E)]
        mx = functools.reduce(jnp.maximum, gm)
        ex = [jnp.exp(gm[e] - mx) for e in range(E)]
        ssum = functools.reduce(lambda a, b: a + b, ex)
        sinv = 1.0 / ssum
        for e in range(E):
            amv[e, sl] = am[e]
            wv[e, sl] = ex[e] * sinv * am[e]
    pltpu.sync_copy(amv, am_hbm.at[:, pl.ds(base, TPW)])
    pltpu.sync_copy(wv, w_hbm.at[:, pl.ds(base, TPW)])


def _qkv_body(x_ref, am_ref, wqkv_ref, rep_ref, q_ref, k_ref, v_ref, *, E, HD):
    x = x_ref[...]
    am = am_ref[...].T
    BN = x.shape[0]
    p = jnp.dot(x.astype(jnp.bfloat16), wqkv_ref[...],
                preferred_element_type=jnp.float32)
    qk = jnp.zeros((BN, 2 * HD), jnp.float32)
    for i in range(E):
        qk = qk + am[:, i:i + 1] * p[:, i * 2 * HD:(i + 1) * 2 * HD]
    q_ref[...] = qk[:, :HD]
    k_ref[...] = qk[:, HD:]
    amr = jnp.dot(am.astype(jnp.bfloat16), rep_ref[...],
                  preferred_element_type=jnp.float32)
    voff = 2 * E * HD
    accv = jnp.zeros((BN, 2 * HD), jnp.float32)
    for j in range(E // 2):
        sl = slice(voff + j * 2 * HD, voff + (j + 1) * 2 * HD)
        accv = accv + p[:, sl] * amr[:, j * 2 * HD:(j + 1) * 2 * HD]
    v_ref[...] = accv[:, :HD] + accv[:, HD:]


def _attn_out_body(q_ref, k_ref, v_ref, w_ref, o_ref, out_ref, *,
                   BQ, T, E, HD, scale):
    qb = pl.program_id(1)
    q = q_ref[...].astype(jnp.bfloat16)
    k = k_ref[...].astype(jnp.bfloat16)
    s = lax.dot_general(q, k, (((1,), (1,)), ((), ())),
                        preferred_element_type=jnp.float32)
    rows = qb * BQ + lax.broadcasted_iota(jnp.int32, (BQ, T), 0)
    cols = lax.broadcasted_iota(jnp.int32, (BQ, T), 1)
    s = jnp.where(cols <= rows, s * scale, -1e9)
    m = jnp.max(s, axis=1, keepdims=True)
    p = jnp.exp(s - m)
    l = jnp.sum(p, axis=1, keepdims=True)
    oh = jnp.dot(p.astype(jnp.bfloat16), v_ref[...].astype(jnp.bfloat16),
                 preferred_element_type=jnp.float32)
    oh = oh / l

    w = w_ref[...].T
    a2 = jnp.concatenate([oh * w[:, i:i + 1] for i in range(E)], axis=1)
    out_ref[...] = jnp.dot(a2.astype(jnp.bfloat16), o_ref[...],
                           preferred_element_type=jnp.float32)


def kernel(hidden_states, sim_matrix, gates, q_proj, k_proj, v_proj, o_proj):
    B, T, C = hidden_states.shape
    E = sim_matrix.shape[1]
    HD = q_proj.shape[2]
    N = B * T
    flat = hidden_states.reshape(N, C)

    wqk = jnp.concatenate([q_proj, k_proj], axis=2)
    wqk = wqk.transpose(1, 0, 2).reshape(C, E * 2 * HD)
    wv = v_proj.transpose(1, 0, 2).reshape(C, E * HD)
    wqkv = jnp.concatenate([wqk, wv], axis=1).astype(jnp.bfloat16)
    rep = (jnp.arange(E * HD)[None, :] // HD
           == jnp.arange(E)[:, None]).astype(jnp.bfloat16)
    o_stack = o_proj.reshape(E * HD, C).astype(jnp.bfloat16)
    gates_row = gates.reshape(1, E)

    BN = 1024 if N % 1024 == 0 else N
    g1 = N // BN
    logits = pl.pallas_call(
        _logits_body,
        grid=(g1,),
        in_specs=[
            pl.BlockSpec((BN, C), lambda i: (i, 0)),
            pl.BlockSpec((C, E), lambda i: (0, 0)),
            pl.BlockSpec((1, E), lambda i: (0, 0)),
        ],
        out_specs=pl.BlockSpec((E, BN), lambda i: (0, i)),
        out_shape=jax.ShapeDtypeStruct((E, N), jnp.float32),
    )(flat, sim_matrix, gates_row)

    NC, NS = 2, 16
    TPW = N // (NC * NS)
    route = pl.kernel(
        functools.partial(_sc_route_body, TPW=TPW, E=E, NC=NC),
        out_type=[jax.ShapeDtypeStruct((E, N), jnp.float32),
                  jax.ShapeDtypeStruct((E, N), jnp.float32)],
        mesh=plsc.VectorSubcoreMesh(core_axis_name="c", subcore_axis_name="s"),
        scratch_types=[pltpu.VMEM((E, TPW), jnp.float32)] * 3,
        compiler_params=pltpu.CompilerParams(needs_layout_passes=False),
    )
    am_t, w_t = route(logits)

    q, k, v = pl.pallas_call(
        functools.partial(_qkv_body, E=E, HD=HD),
        grid=(g1,),
        in_specs=[
            pl.BlockSpec((BN, C), lambda i: (i, 0)),
            pl.BlockSpec((E, BN), lambda i: (0, i)),
            pl.BlockSpec((C, E * 3 * HD), lambda i: (0, 0)),
            pl.BlockSpec((E, E * HD), lambda i: (0, 0)),
        ],
        out_specs=[
            pl.BlockSpec((BN, HD), lambda i: (i, 0)),
            pl.BlockSpec((BN, HD), lambda i: (i, 0)),
            pl.BlockSpec((BN, HD), lambda i: (i, 0)),
        ],
        out_shape=[
            jax.ShapeDtypeStruct((N, HD), jnp.float32),
            jax.ShapeDtypeStruct((N, HD), jnp.float32),
            jax.ShapeDtypeStruct((N, HD), jnp.float32),
        ],
    )(flat, am_t, wqkv, rep)

    qb3 = q.reshape(B, T, HD)
    kb3 = k.reshape(B, T, HD)
    vb3 = v.reshape(B, T, HD)
    wb3 = w_t.reshape(E, B, T).transpose(1, 0, 2)

    BQ = 512 if T % 512 == 0 else T
    scale = 1.0 / float(HD) ** 0.5
    out = pl.pallas_call(
        functools.partial(_attn_out_body, BQ=BQ, T=T, E=E, HD=HD,
                          scale=scale),
        grid=(B, T // BQ),
        in_specs=[
            pl.BlockSpec((None, BQ, HD), lambda b, i: (b, i, 0)),
            pl.BlockSpec((None, T, HD), lambda b, i: (b, 0, 0)),
            pl.BlockSpec((None, T, HD), lambda b, i: (b, 0, 0)),
            pl.BlockSpec((None, E, BQ), lambda b, i: (b, 0, i)),
            pl.BlockSpec((E * HD, C), lambda b, i: (0, 0)),
        ],
        out_specs=pl.BlockSpec((None, BQ, C), lambda b, i: (b, i, 0)),
        out_shape=jax.ShapeDtypeStruct((B, T, C), jnp.float32),
    )(qb3, kb3, vb3, wb3, o_stack)
    return out
